# baseline (speedup 1.0000x reference)
import jax
import jax.numpy as jnp
from jax.experimental import pallas as pl
from jax.experimental.pallas import tpu as pltpu


def _layer_mm(agg3, X, Wr, Wself):
    R, N, D = agg3.shape
    BN = 2000

    def body(x_ref, a_ref, w_ref, ws_ref, o_ref):
        r = pl.program_id(1)

        @pl.when(r == 0)
        def _():
            o_ref[...] = jnp.dot(x_ref[...], ws_ref[...],
                                 preferred_element_type=jnp.float32)

        o_ref[...] += jnp.dot(a_ref[0], w_ref[0],
                              preferred_element_type=jnp.float32)

        @pl.when(r == R - 1)
        def _():
            o_ref[...] = jnp.maximum(o_ref[...], 0.0)

    return pl.pallas_call(
        body,
        grid=(N // BN, R),
        in_specs=[
            pl.BlockSpec((BN, D), lambda i, r: (i, 0)),
            pl.BlockSpec((1, BN, D), lambda i, r: (r, i, 0)),
            pl.BlockSpec((1, D, D), lambda i, r: (r, 0, 0)),
            pl.BlockSpec((D, D), lambda i, r: (0, 0)),
        ],
        out_specs=pl.BlockSpec((BN, D), lambda i, r: (i, 0)),
        out_shape=jax.ShapeDtypeStruct((N, D), jnp.float32),
    )(X, agg3, Wr, Wself)


def kernel(X, edge_index, edge_type, W0, Wself0, W1, Wself1):
    N, D = X.shape
    R = W0.shape[0]
    E = edge_type.shape[0]
    src = edge_index[0].astype(jnp.int32)
    dst = edge_index[1].astype(jnp.int32)
    et = edge_type.astype(jnp.int32)
    comb = et * N + dst
    deg = jax.ops.segment_sum(jnp.ones((E,), jnp.float32), comb,
                              num_segments=N * R)
    invdeg = 1.0 / jnp.clip(deg, 1.0)

    def layer(h, W, Wself):
        msg = h[src]
        agg = jax.ops.segment_sum(msg, comb, num_segments=N * R)
        agg = agg * invdeg[:, None]
        return _layer_mm(agg.reshape(R, N, D), h, W, Wself)

    h = layer(X, W0, Wself0)
    return layer(h, W1, Wself1)
